# Initial kernel scaffold; baseline (speedup 1.0000x reference)
import jax
import jax.numpy as jnp
from jax.experimental import pallas as pl


def kernel(indices, table):
    raise NotImplementedError("write your pallas kernel here")


# speedup vs baseline: 7.4007x; 7.4007x over previous
import functools

import jax
import jax.numpy as jnp
from jax import lax
from jax.experimental import pallas as pl
from jax.experimental.pallas import tpu as pltpu
from jax.experimental.pallas import tpu_sc as plsc

_INFO = plsc.get_sparse_core_info()
_NC = _INFO.num_cores
_NS = _INFO.num_subcores
_NW = _NC * _NS

_CHUNK = 128


@functools.partial(jax.jit, static_argnames=())
def _gather_rows(idx_grouped, table):
    nw, nchunk, chunk = idx_grouped.shape
    n_rows = nw * nchunk * chunk
    d = table.shape[1]
    half = nchunk // 2
    assert nchunk % 2 == 0

    mesh = plsc.VectorSubcoreMesh(core_axis_name="c", subcore_axis_name="s")

    @functools.partial(
        pl.kernel,
        mesh=mesh,
        out_type=jax.ShapeDtypeStruct((n_rows, d), jnp.float32),
        scratch_types=[
            pltpu.VMEM((nchunk, chunk), jnp.int32),
            pltpu.VMEM((chunk, d), jnp.float32),
            pltpu.VMEM((chunk, d), jnp.float32),
            pltpu.SemaphoreType.DMA,
            pltpu.SemaphoreType.DMA,
            pltpu.SemaphoreType.DMA,
            pltpu.SemaphoreType.DMA,
        ],
    )
    def k(idx_hbm, table_hbm, out_hbm, idx_v, rows0, rows1,
          gsem0, gsem1, ssem0, ssem1):
        wid = lax.axis_index("s") * _NC + lax.axis_index("c")
        base = wid * (nchunk * chunk)
        pltpu.sync_copy(idx_hbm.at[wid], idx_v)

        def gcopy(c, rows, sem):
            return pltpu.make_async_copy(table_hbm.at[idx_v.at[c]], rows, sem)

        def scopy(c, rows, sem):
            return pltpu.make_async_copy(
                rows, out_hbm.at[pl.ds(base + c * chunk, chunk)], sem)

        gcopy(0, rows0, gsem0).start()
        gcopy(1, rows1, gsem1).start()

        def body(i, carry):
            c0 = 2 * i
            gcopy(c0, rows0, gsem0).wait()
            scopy(c0, rows0, ssem0).start()
            gcopy(c0 + 1, rows1, gsem1).wait()
            scopy(c0 + 1, rows1, ssem1).start()

            @pl.when(i + 1 < half)
            def _():
                scopy(c0, rows0, ssem0).wait()
                gcopy(c0 + 2, rows0, gsem0).start()
                scopy(c0 + 1, rows1, ssem1).wait()
                gcopy(c0 + 3, rows1, gsem1).start()

            return carry

        lax.fori_loop(0, half, body, 0)
        scopy(nchunk - 2, rows0, ssem0).wait()
        scopy(nchunk - 1, rows1, ssem1).wait()

    return k(idx_grouped, table)


def kernel(indices, table):
    b, l = indices.shape
    d = table.shape[1]
    n = b * l
    rows_per_w = n // _NW
    nchunk = rows_per_w // _CHUNK
    assert rows_per_w % _CHUNK == 0 and n % _NW == 0

    idx_t = jnp.transpose(indices.astype(jnp.int32), (1, 0))
    idx_grouped = idx_t.reshape(_NW, nchunk, _CHUNK)
    out_flat = _gather_rows(idx_grouped, table)
    return out_flat.reshape(l, b, d)


# speedup vs baseline: 8.1301x; 1.0986x over previous
import functools

import jax
import jax.numpy as jnp
from jax import lax
from jax.experimental import pallas as pl
from jax.experimental.pallas import tpu as pltpu
from jax.experimental.pallas import tpu_sc as plsc

_INFO = plsc.get_sparse_core_info()
_NC = _INFO.num_cores
_NS = _INFO.num_subcores
_NW = _NC * _NS

_CHUNK = 128
_NBUF = 5
_LOOKAHEAD = 2


@functools.partial(jax.jit, static_argnames=())
def _gather_rows(idx_grouped, table):
    nw, nchunk, chunk = idx_grouped.shape
    n_rows = nw * nchunk * chunk
    d = table.shape[1]
    nbuf, la = _NBUF, _LOOKAHEAD
    assert nchunk % nbuf == 0

    mesh = plsc.VectorSubcoreMesh(core_axis_name="c", subcore_axis_name="s")

    @functools.partial(
        pl.kernel,
        mesh=mesh,
        out_type=jax.ShapeDtypeStruct((n_rows, d), jnp.float32),
        scratch_types=(
            [pltpu.VMEM((nchunk, chunk), jnp.int32)]
            + [pltpu.VMEM((chunk, d), jnp.float32)] * nbuf
            + [pltpu.SemaphoreType.DMA] * (2 * nbuf)
        ),
    )
    def k(idx_hbm, table_hbm, out_hbm, idx_v, *bufs_and_sems):
        rows = bufs_and_sems[:nbuf]
        gsem = bufs_and_sems[nbuf:2 * nbuf]
        ssem = bufs_and_sems[2 * nbuf:]
        wid = lax.axis_index("s") * _NC + lax.axis_index("c")
        base = wid * (nchunk * chunk)
        pltpu.sync_copy(idx_hbm.at[wid], idx_v)

        def gcopy(c, u):
            return pltpu.make_async_copy(
                table_hbm.at[idx_v.at[c]], rows[u], gsem[u])

        def scopy(c, u):
            return pltpu.make_async_copy(
                rows[u], out_hbm.at[pl.ds(base + c * chunk, chunk)], ssem[u])

        for u in range(la):
            gcopy(u, u).start()

        def body(i, carry):
            cb = i * nbuf
            for u in range(nbuf):
                c = cb + u
                gcopy(c, u).wait()
                scopy(c, u).start()
                cg = c + la
                jg = (u + la) % nbuf

                @pl.when(cg < nchunk)
                def _():
                    @pl.when(cg >= nbuf)
                    def _():
                        scopy(cg - nbuf, jg).wait()
                    gcopy(cg, jg).start()

            return carry

        lax.fori_loop(0, nchunk // nbuf, body, 0)
        for u in range(nbuf):
            scopy(nchunk - nbuf + u, u).wait()

    return k(idx_grouped, table)


def kernel(indices, table):
    b, l = indices.shape
    d = table.shape[1]
    n = b * l
    rows_per_w = n // _NW
    nchunk = rows_per_w // _CHUNK
    assert rows_per_w % _CHUNK == 0 and n % _NW == 0

    idx_t = jnp.transpose(indices.astype(jnp.int32), (1, 0))
    idx_grouped = idx_t.reshape(_NW, nchunk, _CHUNK)
    out_flat = _gather_rows(idx_grouped, table)
    return out_flat.reshape(l, b, d)
